# Initial kernel scaffold; baseline (speedup 1.0000x reference)
import jax
import jax.numpy as jnp
from jax.experimental import pallas as pl


def kernel(x, edge_index, size_factors, W1, b1, ln_g, ln_b, W2, b2):
    raise NotImplementedError("write your pallas kernel here")


# speedup vs baseline: 68.8862x; 68.8862x over previous
import functools

import jax
import jax.numpy as jnp
from jax import lax
from jax.experimental import pallas as pl
from jax.experimental.pallas import tpu as pltpu
from jax.experimental.pallas import tpu_sc as plsc

N = 10000
D = 128
H = 2
E = 320000

NC = 2
NS = 16
NW = NC * NS
EP = E // NW
ECHUNKS = EP // 16
NP = 10112

_MESH = plsc.VectorSubcoreMesh(
    core_axis_name="c", subcore_axis_name="s", num_cores=NC, num_subcores=NS
)
_SC_PARAMS = pltpu.CompilerParams(needs_layout_passes=False)


@functools.partial(
    pl.kernel,
    out_type=jax.ShapeDtypeStruct((NW, NP), jnp.float32),
    mesh=_MESH,
    compiler_params=_SC_PARAMS,
    scratch_types=[
        pltpu.VMEM((EP,), jnp.int32),
        pltpu.VMEM((NP,), jnp.float32),
    ],
)
def _sc_degree(dst_hbm, out_hbm, dst_v, acc_v):
    c = lax.axis_index("c")
    s = lax.axis_index("s")
    wid = s * NC + c
    pltpu.sync_copy(dst_hbm.at[pl.ds(wid * EP, EP)], dst_v)

    zeros16 = jnp.zeros((16,), jnp.float32)

    def zbody(i, _):
        acc_v[pl.ds(i * 16, 16)] = zeros16
        return 0

    lax.fori_loop(0, NP // 16, zbody, 0)

    ones16 = jnp.ones((16,), jnp.float32)

    def ebody(i, _):
        dv = dst_v[pl.ds(i * 16, 16)]
        plsc.addupdate_scatter(acc_v, [dv], ones16)
        return 0

    lax.fori_loop(0, ECHUNKS, ebody, 0)

    pltpu.sync_copy(acc_v, out_hbm.at[wid])


@functools.partial(
    pl.kernel,
    out_type=jax.ShapeDtypeStruct((H * NW, NP), jnp.float32),
    mesh=_MESH,
    compiler_params=_SC_PARAMS,
    scratch_types=[
        pltpu.VMEM((EP,), jnp.int32),
        pltpu.VMEM((EP,), jnp.int32),
        pltpu.VMEM((H * NP,), jnp.float32),
        pltpu.VMEM((H * NP,), jnp.float32),
    ],
)
def _sc_scatter(u_hbm, src_hbm, dst_hbm, out_hbm, src_v, dst_v, u_v, acc_v):
    c = lax.axis_index("c")
    s = lax.axis_index("s")
    wid = s * NC + c
    base = wid * EP
    pltpu.sync_copy(src_hbm.at[pl.ds(base, EP)], src_v)
    pltpu.sync_copy(dst_hbm.at[pl.ds(base, EP)], dst_v)
    pltpu.sync_copy(u_hbm, u_v)

    zeros16 = jnp.zeros((16,), jnp.float32)

    def zbody(i, _):
        acc_v[pl.ds(i * 16, 16)] = zeros16
        return 0

    lax.fori_loop(0, (H * NP) // 16, zbody, 0)

    def ebody(i, _):
        sv = src_v[pl.ds(i * 16, 16)]
        dv = dst_v[pl.ds(i * 16, 16)]
        g0 = plsc.load_gather(u_v, [sv])
        g1 = plsc.load_gather(u_v, [sv + NP])
        plsc.addupdate_scatter(acc_v, [dv], g0)
        plsc.addupdate_scatter(acc_v, [dv + NP], g1)
        return 0

    lax.fori_loop(0, ECHUNKS, ebody, 0)

    pltpu.sync_copy(acc_v.at[pl.ds(0, NP)], out_hbm.at[wid])
    pltpu.sync_copy(acc_v.at[pl.ds(NP, NP)], out_hbm.at[NW + wid])


def _r_deg_body(in_ref, out_ref):
    out_ref[...] = jnp.sum(in_ref[...], axis=0, keepdims=True)


_r_deg = pl.pallas_call(
    _r_deg_body,
    out_shape=jax.ShapeDtypeStruct((1, NP), jnp.float32),
)


def _r_s_body(in_ref, out_ref):
    a = in_ref[...]
    out_ref[0, :] = jnp.sum(a[0:NW], axis=0)
    out_ref[1, :] = jnp.sum(a[NW : 2 * NW], axis=0)


_r_s = pl.pallas_call(
    _r_s_body,
    out_shape=jax.ShapeDtypeStruct((H, NP), jnp.float32),
)


_B = 2000
_GRID = N // _B


def _tc_a_body(x_ref, w1_ref, deg_ref, u1_ref, dinv_ref):
    h1 = jnp.dot(x_ref[...], w1_ref[...], preferred_element_type=jnp.float32)
    dinv = lax.rsqrt(deg_ref[...] + 1.0)
    u1_ref[...] = h1 * dinv
    dinv_ref[...] = dinv


def _tc_b_body(s1_ref, u1_ref, dinv_ref, b1_ref, g_ref, bln_ref, u2_ref):
    dinv = dinv_ref[...]
    a1 = dinv * (s1_ref[...] + u1_ref[...]) + b1_ref[...]
    mu = jnp.mean(a1, axis=-1, keepdims=True)
    var = jnp.mean((a1 - mu) ** 2, axis=-1, keepdims=True)
    z = (a1 - mu) * lax.rsqrt(var + 1e-5) * g_ref[...] + bln_ref[...]
    u2_ref[...] = dinv * jnp.maximum(z, 0.0)


def _tc_c_body(s2_ref, u2_ref, dinv_ref, w2_ref, b2_ref, sf_ref, out_ref):
    a2 = dinv_ref[...] * (s2_ref[...] + u2_ref[...])
    o = jnp.dot(a2, w2_ref[...], preferred_element_type=jnp.float32) + b2_ref[...]
    out_ref[...] = jnp.maximum(o, 0.0) * sf_ref[...]


def _row_spec(last):
    return pl.BlockSpec((_B, last), lambda i: (i, 0))


def _full_spec(shape):
    return pl.BlockSpec(shape, lambda i: tuple(0 for _ in shape))


_tc_a = pl.pallas_call(
    _tc_a_body,
    grid=(_GRID,),
    in_specs=[
        _row_spec(D),
        _full_spec((D, H)),
        _row_spec(1),
    ],
    out_specs=[_row_spec(H), _row_spec(1)],
    out_shape=[
        jax.ShapeDtypeStruct((N, H), jnp.float32),
        jax.ShapeDtypeStruct((N, 1), jnp.float32),
    ],
)

_tc_b = pl.pallas_call(
    _tc_b_body,
    grid=(_GRID,),
    in_specs=[
        _row_spec(H),
        _row_spec(H),
        _row_spec(1),
        _full_spec((H,)),
        _full_spec((H,)),
        _full_spec((H,)),
    ],
    out_specs=[_row_spec(H)],
    out_shape=[jax.ShapeDtypeStruct((N, H), jnp.float32)],
)

_tc_c = pl.pallas_call(
    _tc_c_body,
    grid=(_GRID,),
    in_specs=[
        _row_spec(H),
        _row_spec(H),
        _row_spec(1),
        _full_spec((H, D)),
        _full_spec((D,)),
        _row_spec(1),
    ],
    out_specs=[_row_spec(D)],
    out_shape=[jax.ShapeDtypeStruct((N, D), jnp.float32)],
)


def _to_planar(u):
    return jnp.pad(u.T, ((0, 0), (0, NP - N))).reshape(H * NP)


def kernel(x, edge_index, size_factors, W1, b1, ln_g, ln_b, W2, b2):
    src = edge_index[0]
    dst = edge_index[1]

    degp = _sc_degree(dst)
    deg = _r_deg(degp)[0, :N].reshape(N, 1)
    u1, dinv = _tc_a(x, W1, deg)
    s1p = _sc_scatter(_to_planar(u1), src, dst)
    s1 = _r_s(s1p)[:, :N].T
    (u2,) = _tc_b(s1, u1, dinv, b1, ln_g, ln_b)
    s2p = _sc_scatter(_to_planar(u2), src, dst)
    s2 = _r_s(s2p)[:, :N].T
    (out,) = _tc_c(s2, u2, dinv, W2, b2, size_factors)
    return out


# speedup vs baseline: 76.8803x; 1.1160x over previous
import functools

import jax
import jax.numpy as jnp
from jax import lax
from jax.experimental import pallas as pl
from jax.experimental.pallas import tpu as pltpu
from jax.experimental.pallas import tpu_sc as plsc

N = 10000
D = 128
H = 2
E = 320000

NC = 2
NS = 16
NW = NC * NS
EP = E // NW
ECHUNKS = EP // 16
NP = 10112

_MESH = plsc.VectorSubcoreMesh(
    core_axis_name="c", subcore_axis_name="s", num_cores=NC, num_subcores=NS
)
_SC_PARAMS = pltpu.CompilerParams(needs_layout_passes=False)


@functools.partial(
    pl.kernel,
    out_type=jax.ShapeDtypeStruct((NW, NP), jnp.float32),
    mesh=_MESH,
    compiler_params=_SC_PARAMS,
    scratch_types=[
        pltpu.VMEM((EP,), jnp.int32),
        pltpu.VMEM((NP,), jnp.float32),
    ],
)
def _sc_degree(dst_hbm, z_hbm, out_hbm, dst_v, acc_v):
    c = lax.axis_index("c")
    s = lax.axis_index("s")
    wid = s * NC + c
    pltpu.sync_copy(dst_hbm.at[pl.ds(wid * EP, EP)], dst_v)
    pltpu.sync_copy(z_hbm.at[pl.ds(0, NP)], acc_v)

    ones16 = jnp.ones((16,), jnp.float32)

    @plsc.parallel_loop(0, ECHUNKS, unroll=8)
    def ebody(i):
        dv = dst_v[pl.ds(i * 16, 16)]
        plsc.addupdate_scatter(acc_v, [dv], ones16)

    pltpu.sync_copy(acc_v, out_hbm.at[wid])


@functools.partial(
    pl.kernel,
    out_type=jax.ShapeDtypeStruct((H * NW, NP), jnp.float32),
    mesh=_MESH,
    compiler_params=_SC_PARAMS,
    scratch_types=[
        pltpu.VMEM((EP,), jnp.int32),
        pltpu.VMEM((EP,), jnp.int32),
        pltpu.VMEM((H * NP,), jnp.float32),
        pltpu.VMEM((H * NP,), jnp.float32),
    ],
)
def _sc_scatter(u_hbm, src_hbm, dst_hbm, z_hbm, out_hbm, src_v, dst_v, u_v, acc_v):
    c = lax.axis_index("c")
    s = lax.axis_index("s")
    wid = s * NC + c
    base = wid * EP
    pltpu.sync_copy(src_hbm.at[pl.ds(base, EP)], src_v)
    pltpu.sync_copy(dst_hbm.at[pl.ds(base, EP)], dst_v)
    pltpu.sync_copy(u_hbm, u_v)
    pltpu.sync_copy(z_hbm, acc_v)

    @plsc.parallel_loop(0, ECHUNKS, unroll=8)
    def ebody(i):
        sv = src_v[pl.ds(i * 16, 16)]
        dv = dst_v[pl.ds(i * 16, 16)]
        g0 = plsc.load_gather(u_v, [sv])
        g1 = plsc.load_gather(u_v, [sv + NP])
        plsc.addupdate_scatter(acc_v, [dv], g0)
        plsc.addupdate_scatter(acc_v, [dv + NP], g1)

    pltpu.sync_copy(acc_v.at[pl.ds(0, NP)], out_hbm.at[wid])
    pltpu.sync_copy(acc_v.at[pl.ds(NP, NP)], out_hbm.at[NW + wid])


def _r_deg_body(in_ref, out_ref):
    out_ref[...] = jnp.sum(in_ref[...], axis=0, keepdims=True)


_r_deg = pl.pallas_call(
    _r_deg_body,
    out_shape=jax.ShapeDtypeStruct((1, NP), jnp.float32),
)


def _r_s_body(in_ref, out_ref):
    a = in_ref[...]
    out_ref[0, :] = jnp.sum(a[0:NW], axis=0)
    out_ref[1, :] = jnp.sum(a[NW : 2 * NW], axis=0)


_r_s = pl.pallas_call(
    _r_s_body,
    out_shape=jax.ShapeDtypeStruct((H, NP), jnp.float32),
)


_B = 2000
_GRID = N // _B


def _tc_a_body(x_ref, w1_ref, deg_ref, u1_ref, dinv_ref):
    h1 = jnp.dot(x_ref[...], w1_ref[...], preferred_element_type=jnp.float32)
    dinv = lax.rsqrt(deg_ref[...] + 1.0)
    u1_ref[...] = h1 * dinv
    dinv_ref[...] = dinv


def _tc_b_body(s1_ref, u1_ref, dinv_ref, b1_ref, g_ref, bln_ref, u2_ref):
    dinv = dinv_ref[...]
    a1 = dinv * (s1_ref[...] + u1_ref[...]) + b1_ref[...]
    mu = jnp.mean(a1, axis=-1, keepdims=True)
    var = jnp.mean((a1 - mu) ** 2, axis=-1, keepdims=True)
    z = (a1 - mu) * lax.rsqrt(var + 1e-5) * g_ref[...] + bln_ref[...]
    u2_ref[...] = dinv * jnp.maximum(z, 0.0)


def _tc_c_body(s2_ref, u2_ref, dinv_ref, w2_ref, b2_ref, sf_ref, out_ref):
    a2 = dinv_ref[...] * (s2_ref[...] + u2_ref[...])
    o = jnp.dot(a2, w2_ref[...], preferred_element_type=jnp.float32) + b2_ref[...]
    out_ref[...] = jnp.maximum(o, 0.0) * sf_ref[...]


def _row_spec(last):
    return pl.BlockSpec((_B, last), lambda i: (i, 0))


def _full_spec(shape):
    return pl.BlockSpec(shape, lambda i: tuple(0 for _ in shape))


_tc_a = pl.pallas_call(
    _tc_a_body,
    grid=(_GRID,),
    in_specs=[
        _row_spec(D),
        _full_spec((D, H)),
        _row_spec(1),
    ],
    out_specs=[_row_spec(H), _row_spec(1)],
    out_shape=[
        jax.ShapeDtypeStruct((N, H), jnp.float32),
        jax.ShapeDtypeStruct((N, 1), jnp.float32),
    ],
)

_tc_b = pl.pallas_call(
    _tc_b_body,
    grid=(_GRID,),
    in_specs=[
        _row_spec(H),
        _row_spec(H),
        _row_spec(1),
        _full_spec((H,)),
        _full_spec((H,)),
        _full_spec((H,)),
    ],
    out_specs=[_row_spec(H)],
    out_shape=[jax.ShapeDtypeStruct((N, H), jnp.float32)],
)

_tc_c = pl.pallas_call(
    _tc_c_body,
    grid=(_GRID,),
    in_specs=[
        _row_spec(H),
        _row_spec(H),
        _row_spec(1),
        _full_spec((H, D)),
        _full_spec((D,)),
        _row_spec(1),
    ],
    out_specs=[_row_spec(D)],
    out_shape=[jax.ShapeDtypeStruct((N, D), jnp.float32)],
)


def _to_planar(u):
    return jnp.pad(u.T, ((0, 0), (0, NP - N))).reshape(H * NP)


def kernel(x, edge_index, size_factors, W1, b1, ln_g, ln_b, W2, b2):
    src = edge_index[0]
    dst = edge_index[1]

    zeros_hbm = jnp.zeros((H * NP,), jnp.float32)
    degp = _sc_degree(dst, zeros_hbm)
    deg = _r_deg(degp)[0, :N].reshape(N, 1)
    u1, dinv = _tc_a(x, W1, deg)
    s1p = _sc_scatter(_to_planar(u1), src, dst, zeros_hbm)
    s1 = _r_s(s1p)[:, :N].T
    (u2,) = _tc_b(s1, u1, dinv, b1, ln_g, ln_b)
    s2p = _sc_scatter(_to_planar(u2), src, dst, zeros_hbm)
    s2 = _r_s(s2p)[:, :N].T
    (out,) = _tc_c(s2, u2, dinv, W2, b2, size_factors)
    return out


# speedup vs baseline: 99.1711x; 1.2899x over previous
import functools

import jax
import jax.numpy as jnp
from jax import lax
from jax.experimental import pallas as pl
from jax.experimental.pallas import tpu as pltpu
from jax.experimental.pallas import tpu_sc as plsc

N = 10000
D = 128
H = 2
E = 320000

NC = 2
NS = 16
NW = NC * NS
EP = E // NW
ECHUNKS = EP // 16
NP = 10112

_MESH = plsc.VectorSubcoreMesh(
    core_axis_name="c", subcore_axis_name="s", num_cores=NC, num_subcores=NS
)
_SC_PARAMS = pltpu.CompilerParams(needs_layout_passes=False)


@functools.partial(
    pl.kernel,
    out_type=jax.ShapeDtypeStruct((NW, NP), jnp.float32),
    mesh=_MESH,
    compiler_params=_SC_PARAMS,
    scratch_types=[
        pltpu.VMEM((EP,), jnp.int32),
        pltpu.VMEM((NP,), jnp.float32),
    ],
)
def _sc_degree(dst_hbm, z_hbm, out_hbm, dst_v, acc_v):
    c = lax.axis_index("c")
    s = lax.axis_index("s")
    wid = s * NC + c
    pltpu.sync_copy(dst_hbm.at[pl.ds(wid * EP, EP)], dst_v)
    pltpu.sync_copy(z_hbm.at[pl.ds(0, NP)], acc_v)

    ones16 = jnp.ones((16,), jnp.float32)

    @plsc.parallel_loop(0, ECHUNKS, unroll=8)
    def ebody(i):
        dv = dst_v[pl.ds(i * 16, 16)]
        plsc.addupdate_scatter(acc_v, [dv], ones16)

    pltpu.sync_copy(acc_v, out_hbm.at[wid])


@functools.partial(
    pl.kernel,
    out_type=jax.ShapeDtypeStruct((H * NW, NP), jnp.float32),
    mesh=_MESH,
    compiler_params=_SC_PARAMS,
    scratch_types=[
        pltpu.VMEM((EP,), jnp.int32),
        pltpu.VMEM((EP,), jnp.int32),
        pltpu.VMEM((H * NP,), jnp.float32),
        pltpu.VMEM((H * NP,), jnp.float32),
    ],
)
def _sc_scatter(u_hbm, src_hbm, dst_hbm, z_hbm, out_hbm, src_v, dst_v, u_v, acc_v):
    c = lax.axis_index("c")
    s = lax.axis_index("s")
    wid = s * NC + c
    base = wid * EP
    pltpu.sync_copy(src_hbm.at[pl.ds(base, EP)], src_v)
    pltpu.sync_copy(dst_hbm.at[pl.ds(base, EP)], dst_v)
    pltpu.sync_copy(u_hbm, u_v)
    pltpu.sync_copy(z_hbm, acc_v)

    @plsc.parallel_loop(0, ECHUNKS, unroll=8)
    def ebody(i):
        sv = src_v[pl.ds(i * 16, 16)]
        dv = dst_v[pl.ds(i * 16, 16)]
        g0 = plsc.load_gather(u_v, [sv])
        g1 = plsc.load_gather(u_v, [sv + NP])
        plsc.addupdate_scatter(acc_v, [dv], g0)
        plsc.addupdate_scatter(acc_v, [dv + NP], g1)

    pltpu.sync_copy(acc_v.at[pl.ds(0, NP)], out_hbm.at[wid])
    pltpu.sync_copy(acc_v.at[pl.ds(NP, NP)], out_hbm.at[NW + wid])


def _r_deg_body(in_ref, out_ref):
    out_ref[...] = jnp.sum(in_ref[...], axis=0, keepdims=True)


_r_deg = pl.pallas_call(
    _r_deg_body,
    out_shape=jax.ShapeDtypeStruct((1, NP), jnp.float32),
)


_B = 2000
_GRID = N // _B


def _tc_a_body(x_ref, w1_ref, deg_ref, u1_ref):
    h1 = jnp.dot(x_ref[...], w1_ref[...], preferred_element_type=jnp.float32)
    u1_ref[...] = h1 * lax.rsqrt(deg_ref[...] + 1.0)


_tc_a = pl.pallas_call(
    _tc_a_body,
    grid=(_GRID,),
    in_specs=[
        pl.BlockSpec((_B, D), lambda i: (i, 0)),
        pl.BlockSpec((D, H), lambda i: (0, 0)),
        pl.BlockSpec((_B, 1), lambda i: (i, 0)),
    ],
    out_specs=[pl.BlockSpec((_B, H), lambda i: (i, 0))],
    out_shape=[jax.ShapeDtypeStruct((N, H), jnp.float32)],
)


def _tc_b_body(s1p_ref, degp_ref, u1p_ref, b1_ref, g_ref, bln_ref, u2p_ref):
    deg = jnp.sum(degp_ref[...], axis=0, keepdims=True) + 1.0
    dinv = lax.rsqrt(deg)
    s1_0 = jnp.sum(s1p_ref[0:NW], axis=0, keepdims=True)
    s1_1 = jnp.sum(s1p_ref[NW : 2 * NW], axis=0, keepdims=True)
    a1_0 = dinv * (s1_0 + u1p_ref[0:1]) + b1_ref[0]
    a1_1 = dinv * (s1_1 + u1p_ref[1:2]) + b1_ref[1]
    d = 0.5 * (a1_0 - a1_1)
    r = lax.rsqrt(d * d + 1e-5)
    z0 = d * r * g_ref[0] + bln_ref[0]
    z1 = -d * r * g_ref[1] + bln_ref[1]
    u2p_ref[0:1] = dinv * jnp.maximum(z0, 0.0)
    u2p_ref[1:2] = dinv * jnp.maximum(z1, 0.0)


_tc_b = pl.pallas_call(
    _tc_b_body,
    in_specs=[
        pl.BlockSpec(memory_space=pltpu.VMEM),
        pl.BlockSpec(memory_space=pltpu.VMEM),
        pl.BlockSpec(memory_space=pltpu.VMEM),
        pl.BlockSpec(memory_space=pltpu.SMEM),
        pl.BlockSpec(memory_space=pltpu.SMEM),
        pl.BlockSpec(memory_space=pltpu.SMEM),
    ],
    out_shape=jax.ShapeDtypeStruct((H, NP), jnp.float32),
)


def _tc_c_body(s2p_ref, degp_ref, u2p_ref, w2_ref, b2_ref, sfp_ref, out_ref):
    deg = jnp.sum(degp_ref[...], axis=0, keepdims=True) + 1.0
    dinv = lax.rsqrt(deg)
    s2_0 = jnp.sum(s2p_ref[0:NW], axis=0, keepdims=True)
    s2_1 = jnp.sum(s2p_ref[NW : 2 * NW], axis=0, keepdims=True)
    sf = sfp_ref[...]
    a2_0 = sf * dinv * (s2_0 + u2p_ref[0:1])
    a2_1 = sf * dinv * (s2_1 + u2p_ref[1:2])
    lhs = jnp.concatenate(
        [a2_0, a2_1, sf, jnp.zeros((5, NP), jnp.float32)], axis=0
    )
    rhs = jnp.concatenate(
        [w2_ref[...], b2_ref[...].reshape(1, D), jnp.zeros((5, D), jnp.float32)],
        axis=0,
    )
    o = lax.dot_general(
        lhs, rhs, (((0,), (0,)), ((), ())), preferred_element_type=jnp.float32
    )
    out_ref[...] = jnp.maximum(o[:N], 0.0)


_tc_c = pl.pallas_call(
    _tc_c_body,
    out_shape=jax.ShapeDtypeStruct((N, D), jnp.float32),
)


def kernel(x, edge_index, size_factors, W1, b1, ln_g, ln_b, W2, b2):
    src = edge_index[0]
    dst = edge_index[1]

    zeros_hbm = jnp.zeros((H * NP,), jnp.float32)
    degp = _sc_degree(dst, zeros_hbm)
    deg_rm = _r_deg(degp)[0, :N].reshape(N, 1)
    (u1,) = _tc_a(x, W1, deg_rm)
    u1p = jnp.pad(u1.T, ((0, 0), (0, NP - N)))
    s1p = _sc_scatter(u1p.reshape(H * NP), src, dst, zeros_hbm)
    u2p = _tc_b(s1p, degp, u1p, b1, ln_g, ln_b)
    s2p = _sc_scatter(u2p.reshape(H * NP), src, dst, zeros_hbm)
    sfp = jnp.pad(size_factors.T, ((0, 0), (0, NP - N)))
    out = _tc_c(s2p, degp, u2p, W2, b2, sfp)
    return out
